# Initial kernel scaffold; baseline (speedup 1.0000x reference)
import jax
import jax.numpy as jnp
from jax.experimental import pallas as pl


def kernel(species, coordinates, conv_tensor):
    raise NotImplementedError("write your pallas kernel here")


# speedup vs baseline: 184.5694x; 184.5694x over previous
import functools

import jax
import jax.numpy as jnp
from jax import lax
from jax.experimental import pallas as pl
from jax.experimental.pallas import tpu as pltpu
from jax.experimental.pallas import tpu_sc as plsc

_L = 16
_CHUNK = 12800


@functools.partial(jax.jit, static_argnames=())
def _sc_convert(species_flat, conv_tensor):
    n = species_flat.shape[0]
    info = plsc.get_sparse_core_info()
    nc, ns = info.num_cores, info.num_subcores
    nw = nc * ns
    per_w = n // nw
    assert per_w * nw == n and per_w % _CHUNK == 0
    n_chunks = per_w // _CHUNK
    table_n = conv_tensor.shape[0]

    mesh = plsc.VectorSubcoreMesh(core_axis_name="c", subcore_axis_name="s")

    @functools.partial(
        pl.kernel,
        mesh=mesh,
        compiler_params=pltpu.CompilerParams(needs_layout_passes=False),
        out_type=jax.ShapeDtypeStruct((n,), jnp.int32),
        scratch_types=[
            pltpu.VMEM((table_n,), jnp.int32),
            pltpu.VMEM((_CHUNK,), jnp.int32),
            pltpu.VMEM((_CHUNK,), jnp.int32),
        ],
    )
    def k(species_hbm, conv_hbm, out_hbm, table_v, in_v, out_v):
        wid = lax.axis_index("s") * nc + lax.axis_index("c")
        pltpu.sync_copy(conv_hbm, table_v)
        base0 = wid * per_w

        def chunk_body(ci, carry):
            base = base0 + ci * _CHUNK
            pltpu.sync_copy(species_hbm.at[pl.ds(base, _CHUNK)], in_v)

            def body(i, c):
                idx = in_v[pl.ds(i * _L, _L)]
                out_v[pl.ds(i * _L, _L)] = plsc.load_gather(table_v, [idx])
                return c

            lax.fori_loop(0, _CHUNK // _L, body, 0)
            pltpu.sync_copy(out_v, out_hbm.at[pl.ds(base, _CHUNK)])
            return carry

        lax.fori_loop(0, n_chunks, chunk_body, 0)

    return k(species_flat, conv_tensor)


def kernel(species, coordinates, conv_tensor):
    converted = _sc_convert(species.reshape(-1), conv_tensor)
    return converted.reshape(species.shape), coordinates
